# baseline (speedup 1.0000x reference)
import jax
import jax.numpy as jnp
from jax import lax
from jax.experimental import pallas as pl
from jax.experimental.pallas import tpu as pltpu
from jax.experimental.pallas import tpu_sc as plsc

N = 100000
NUM_SEG = 512
D = 3
KDIM = 16
NC = 2
NS = 16
NW = NC * NS
BN = 14336
NPAD = 7 * BN
NMAIN = N - 32
CHUNK = NPAD // NW
NBIN = NUM_SEG + 16
BINS_PER_W = NUM_SEG // NS
_SIZES = [BN] * 6 + [NMAIN - 6 * BN]


def _tc_elem_body(a_ref, c1_ref, c2_ref, tT, xpT, xT, ohT, p0T,
                  lc_ref, ld_ref,
                  t_b, xp_b, x_b, oh_b, p0_b, lc_b, ld_b,
                  sin0, sin1, sout0, sout1):
    a = a_ref[0, 0]
    c1 = c1_ref[0, 0]
    c2 = c2_ref[0, 0]
    sin = (sin0, sin1)
    sout = (sout0, sout1)

    def fire_in(i):
        b = i % 2
        sz = _SIZES[i]
        col = pl.ds(i * BN, sz)
        dst = pl.ds(0, sz)
        return [
            pltpu.async_copy(tT.at[:, col], t_b.at[b, :, dst], sin[b]),
            pltpu.async_copy(xpT.at[:, col], xp_b.at[b, :, dst], sin[b]),
            pltpu.async_copy(xT.at[:, col], x_b.at[b, :, dst], sin[b]),
            pltpu.async_copy(ohT.at[:, col], oh_b.at[b, :, dst], sin[b]),
            pltpu.async_copy(p0T.at[:, col], p0_b.at[b, :, dst], sin[b]),
        ]

    pending_in = fire_in(0)
    pending_out = [None, None]
    for i in range(len(_SIZES)):
        b = i % 2
        sz = _SIZES[i]
        nxt = fire_in(i + 1) if i + 1 < len(_SIZES) else None
        for h in pending_in:
            h.wait()
        if pending_out[b] is not None:
            for h in pending_out[b]:
                h.wait()
        cols = pl.ds(0, sz)
        tv = t_b[b, 0, cols]
        dx = xp_b[b, :, cols] - x_b[b, :, cols]
        se = jnp.sum(dx * dx, axis=0)
        lc_b[b, 0, cols] = c1 * jnp.exp(a * tv) * se
        dq = oh_b[b, :, cols] - p0_b[b, :, cols]
        se2 = jnp.sum(dq * dq, axis=0)
        ld_b[b, 0, cols] = c2 * tv * se2
        col = pl.ds(i * BN, sz)
        pending_out[b] = [
            pltpu.async_copy(lc_b.at[b, :, cols], lc_ref.at[:, col], sout[b]),
            pltpu.async_copy(ld_b.at[b, :, cols], ld_ref.at[:, col], sout[b]),
        ]
        pending_in = nxt
    for po in pending_out:
        if po is not None:
            for h in po:
                h.wait()


@jax.jit
def _tc_elem(a, c1, c2, tT, xpT, xT, ohT, p0T):
    smem = pl.BlockSpec(memory_space=pltpu.SMEM)
    anys = pl.BlockSpec(memory_space=pl.ANY)
    return pl.pallas_call(
        _tc_elem_body,
        in_specs=[smem, smem, smem, anys, anys, anys, anys, anys],
        out_specs=[anys, anys],
        out_shape=[
            jax.ShapeDtypeStruct((1, NPAD), jnp.float32),
            jax.ShapeDtypeStruct((1, NPAD), jnp.float32),
        ],
        scratch_shapes=[
            pltpu.VMEM((2, 1, BN), jnp.float32),
            pltpu.VMEM((2, D, BN), jnp.float32),
            pltpu.VMEM((2, D, BN), jnp.float32),
            pltpu.VMEM((2, KDIM, BN), jnp.float32),
            pltpu.VMEM((2, KDIM, BN), jnp.float32),
            pltpu.VMEM((2, 1, BN), jnp.float32),
            pltpu.VMEM((2, 1, BN), jnp.float32),
            pltpu.SemaphoreType.DMA,
            pltpu.SemaphoreType.DMA,
            pltpu.SemaphoreType.DMA,
            pltpu.SemaphoreType.DMA,
        ],
    )(a, c1, c2, tT, xpT, xT, ohT, p0T)


STRIDE = 197
NSTEP = 197
LAST_CHUNK = N - (NW - 1) * CHUNK
TAIL_OFF = NMAIN - (NW - 1) * CHUNK


def _sc_body(lc_hbm, ld_hbm, ids_hbm, lct_hbm, ldt_hbm, out_hbm,
             lc_v, ld_v, ids_v, acc_c, acc_d, acc_n,
             res0, res1, res2, shared, rbuf, sem_a, sem_b0, sem_b1):
    c = lax.axis_index("c")
    s = lax.axis_index("s")
    wid = c * NS + s
    base = wid * CHUNK

    h_lc = pltpu.async_copy(lc_hbm.at[pl.ds(base, CHUNK)], lc_v, sem_a)
    h_ld = pltpu.async_copy(ld_hbm.at[pl.ds(base, CHUNK)], ld_v, sem_a)

    @pl.when(wid < NW - 1)
    def _ids_full():
        pltpu.sync_copy(ids_hbm.at[pl.ds(base, CHUNK)], ids_v)

    @pl.when(wid == NW - 1)
    def _ids_tail():
        pltpu.sync_copy(ids_hbm.at[pl.ds(base, LAST_CHUNK)],
                        ids_v.at[pl.ds(0, LAST_CHUNK)])
        seg16 = jnp.full((16,), NUM_SEG, jnp.int32)
        for k in range((CHUNK - LAST_CHUNK) // 16):
            ids_v[pl.ds(LAST_CHUNK + k * 16, 16)] = seg16

    zeros16 = jnp.zeros((16,), jnp.float32)
    for h in range(NBIN // 16):
        acc_c[pl.ds(h * 16, 16)] = zeros16
        acc_d[pl.ds(h * 16, 16)] = zeros16
        acc_n[pl.ds(h * 16, 16)] = zeros16

    h_lc.wait()
    h_ld.wait()

    @pl.when(wid == NW - 1)
    def _tail_losses():
        pltpu.sync_copy(lct_hbm, lc_v.at[pl.ds(TAIL_OFF, 32)])
        pltpu.sync_copy(ldt_hbm, ld_v.at[pl.ds(TAIL_OFF, 32)])

    ones16 = jnp.full((16,), 1.0, jnp.float32)
    lane_base = lax.iota(jnp.int32, 16) * STRIDE
    limit16 = jnp.full((16,), CHUNK - 1, jnp.int32)

    def step(j, carry):
        idx = lane_base + j
        valid = idx < CHUNK
        idxc = jnp.minimum(idx, limit16)
        ids = plsc.load_gather(ids_v, [idxc])
        lcv = plsc.load_gather(lc_v, [idxc])
        ldv = plsc.load_gather(ld_v, [idxc])
        plsc.addupdate_scatter(acc_c, [ids], lcv, mask=valid)
        plsc.addupdate_scatter(acc_d, [ids], ldv, mask=valid)
        plsc.addupdate_scatter(acc_n, [ids], ones16, mask=valid)
        return carry

    lax.fori_loop(0, NSTEP, step, 0)

    pltpu.sync_copy(acc_c.at[pl.ds(0, NUM_SEG)], shared.at[0, s, 0])
    pltpu.sync_copy(acc_d.at[pl.ds(0, NUM_SEG)], shared.at[1, s, 0])
    pltpu.sync_copy(acc_n.at[pl.ds(0, NUM_SEG)], shared.at[2, s, 0])
    plsc.subcore_barrier()

    sems = (sem_b0, sem_b1)

    def fire(v, b):
        return [
            pltpu.async_copy(
                shared.at[a, v, 0, pl.ds(s * BINS_PER_W, BINS_PER_W)],
                rbuf.at[a, b, 0], sems[b])
            for a in range(3)
        ]

    accs = [[zeros16 for _ in range(BINS_PER_W // 16)] for _ in range(3)]
    pending = fire(0, 0)
    for v in range(NS):
        b = v % 2
        nxt = fire(v + 1, (v + 1) % 2) if v + 1 < NS else None
        for h_ in pending:
            h_.wait()
        for a in range(3):
            for h in range(BINS_PER_W // 16):
                accs[a][h] = accs[a][h] + rbuf[a, b, 0, pl.ds(h * 16, 16)]
        pending = nxt
    for a, res in ((0, res0), (1, res1), (2, res2)):
        for h in range(BINS_PER_W // 16):
            res[pl.ds(h * 16, 16)] = accs[a][h]
    for a, res in ((0, res0), (1, res1), (2, res2)):
        pltpu.sync_copy(
            res,
            out_hbm.at[pl.ds(c * (3 * NUM_SEG) + a * NUM_SEG + s * BINS_PER_W,
                             BINS_PER_W)])


@jax.jit
def _sc_call(lc, ld, ids, lc_tail, ld_tail):
    mesh = plsc.VectorSubcoreMesh(core_axis_name="c", subcore_axis_name="s")
    return pl.kernel(
        _sc_body,
        out_type=jax.ShapeDtypeStruct((NC * 3 * NUM_SEG,), jnp.float32),
        mesh=mesh,
        compiler_params=pltpu.CompilerParams(needs_layout_passes=False),
        scratch_types=[
            pltpu.VMEM((CHUNK,), jnp.float32),
            pltpu.VMEM((CHUNK,), jnp.float32),
            pltpu.VMEM((CHUNK,), jnp.int32),
            pltpu.VMEM((NBIN,), jnp.float32),
            pltpu.VMEM((NBIN,), jnp.float32),
            pltpu.VMEM((NBIN,), jnp.float32),
            pltpu.VMEM((BINS_PER_W,), jnp.float32),
            pltpu.VMEM((BINS_PER_W,), jnp.float32),
            pltpu.VMEM((BINS_PER_W,), jnp.float32),
            pltpu.VMEM_SHARED((3, NS, 1, NUM_SEG), jnp.float32),
            pltpu.VMEM((3, 2, 1, BINS_PER_W), jnp.float32),
            pltpu.SemaphoreType.DMA,
            pltpu.SemaphoreType.DMA,
            pltpu.SemaphoreType.DMA,
        ],
    )(lc, ld, ids, lc_tail, ld_tail)


def _tc_epilogue_body(p_ref, o_ref):
    s0 = p_ref[pl.ds(0, NUM_SEG)] + p_ref[pl.ds(3 * NUM_SEG, NUM_SEG)]
    s1 = p_ref[pl.ds(NUM_SEG, NUM_SEG)] + p_ref[pl.ds(4 * NUM_SEG, NUM_SEG)]
    s2 = p_ref[pl.ds(2 * NUM_SEG, NUM_SEG)] + p_ref[pl.ds(5 * NUM_SEG, NUM_SEG)]
    cnt = jnp.maximum(s2, 1.0)
    o_ref[...] = jnp.stack([s0 / cnt, s1 / cnt])


@jax.jit
def _tc_epilogue(partial):
    return pl.pallas_call(
        _tc_epilogue_body,
        out_shape=jax.ShapeDtypeStruct((2, NUM_SEG), jnp.float32),
    )(partial)


def kernel(t, sigma1, x_pred, x, segment_ids, beta1, one_hot_x, p_0, K):
    ln_s = jnp.log(sigma1[0])
    a_s = -2.0 * ln_s
    c1_s = -ln_s
    c2_s = K * beta1[0]
    lc2, ld2 = _tc_elem(jnp.reshape(a_s, (1, 1)), jnp.reshape(c1_s, (1, 1)),
                        jnp.reshape(c2_s, (1, 1)),
                        t.T, x_pred.T, x.T, one_hot_x.T, p_0.T)
    tl = t[NMAIN:, 0]
    se_t = jnp.sum((x_pred[NMAIN:] - x[NMAIN:]) ** 2, axis=1)
    lc_tail = c1_s * jnp.exp(a_s * tl) * se_t
    se2_t = jnp.sum((one_hot_x[NMAIN:] - p_0[NMAIN:]) ** 2, axis=1)
    ld_tail = c2_s * tl * se2_t
    partial = _sc_call(lc2.reshape(-1), ld2.reshape(-1),
                       segment_ids.astype(jnp.int32), lc_tail, ld_tail)
    return _tc_epilogue(partial)


# speedup vs baseline: 1.0860x; 1.0860x over previous
import functools

import jax
import jax.numpy as jnp
from jax import lax
from jax.experimental import pallas as pl
from jax.experimental.pallas import tpu as pltpu
from jax.experimental.pallas import tpu_sc as plsc

N = 100000
NUM_SEG = 512
D = 3
KDIM = 16
NC = 2
NS = 16
NW = NC * NS
BN = 14336
NPAD = 7 * BN
CHUNK = NPAD // NW
NVEC = CHUNK // 16
NBIN = NUM_SEG + 16
BINS_PER_W = NUM_SEG // NS


def _tc_elem_body(a_ref, c1_ref, c2_ref, t_ref, xp_ref, x_ref, oh_ref,
                  p0_ref, lc_ref, ld_ref):
    a = a_ref[0, 0]
    c1 = c1_ref[0, 0]
    c2 = c2_ref[0, 0]
    tv = t_ref[0, :]
    dx = xp_ref[...] - x_ref[...]
    se = jnp.sum(dx * dx, axis=0)
    lc_ref[...] = c1 * jnp.exp(a * tv) * se
    dq = oh_ref[...] - p0_ref[...]
    se2 = jnp.sum(dq * dq, axis=0)
    ld_ref[...] = c2 * tv * se2


@jax.jit
def _tc_elem(a, c1, c2, tT, xpT, xT, ohT, p0T):
    smem = pl.BlockSpec(memory_space=pltpu.SMEM)
    return pl.pallas_call(
        _tc_elem_body,
        grid=(NPAD // BN,),
        in_specs=[
            smem, smem, smem,
            pl.BlockSpec((1, BN), lambda i: (0, i)),
            pl.BlockSpec((D, BN), lambda i: (0, i)),
            pl.BlockSpec((D, BN), lambda i: (0, i)),
            pl.BlockSpec((KDIM, BN), lambda i: (0, i)),
            pl.BlockSpec((KDIM, BN), lambda i: (0, i)),
        ],
        out_specs=[
            pl.BlockSpec((BN,), lambda i: (i,)),
            pl.BlockSpec((BN,), lambda i: (i,)),
        ],
        out_shape=[
            jax.ShapeDtypeStruct((NPAD,), jnp.float32),
            jax.ShapeDtypeStruct((NPAD,), jnp.float32),
        ],
    )(a, c1, c2, tT, xpT, xT, ohT, p0T)


STRIDE = 197
NSTEP = 197
LAST_CHUNK = N - (NW - 1) * CHUNK


def _sc_body(lc_hbm, ld_hbm, ids_hbm, out_hbm,
             lc_v, ld_v, ids_v, acc_c, acc_d, acc_n,
             res0, res1, res2, shared, rbuf, sem_a, sem_b0, sem_b1):
    c = lax.axis_index("c")
    s = lax.axis_index("s")
    wid = c * NS + s
    base = wid * CHUNK

    h_lc = pltpu.async_copy(lc_hbm.at[pl.ds(base, CHUNK)], lc_v, sem_a)
    h_ld = pltpu.async_copy(ld_hbm.at[pl.ds(base, CHUNK)], ld_v, sem_a)

    @pl.when(wid < NW - 1)
    def _ids_full():
        pltpu.sync_copy(ids_hbm.at[pl.ds(base, CHUNK)], ids_v)

    @pl.when(wid == NW - 1)
    def _ids_tail():
        pltpu.sync_copy(ids_hbm.at[pl.ds(base, LAST_CHUNK)],
                        ids_v.at[pl.ds(0, LAST_CHUNK)])
        seg16 = jnp.full((16,), NUM_SEG, jnp.int32)
        for k in range((CHUNK - LAST_CHUNK) // 16):
            ids_v[pl.ds(LAST_CHUNK + k * 16, 16)] = seg16

    zeros16 = jnp.zeros((16,), jnp.float32)
    for h in range(NBIN // 16):
        acc_c[pl.ds(h * 16, 16)] = zeros16
        acc_d[pl.ds(h * 16, 16)] = zeros16
        acc_n[pl.ds(h * 16, 16)] = zeros16

    h_lc.wait()
    h_ld.wait()

    ones16 = jnp.full((16,), 1.0, jnp.float32)
    lane_base = lax.iota(jnp.int32, 16) * STRIDE
    limit16 = jnp.full((16,), CHUNK - 1, jnp.int32)

    def step(j, carry):
        idx = lane_base + j
        valid = idx < CHUNK
        idxc = jnp.minimum(idx, limit16)
        ids = plsc.load_gather(ids_v, [idxc])
        lcv = plsc.load_gather(lc_v, [idxc])
        ldv = plsc.load_gather(ld_v, [idxc])
        plsc.addupdate_scatter(acc_c, [ids], lcv, mask=valid)
        plsc.addupdate_scatter(acc_d, [ids], ldv, mask=valid)
        plsc.addupdate_scatter(acc_n, [ids], ones16, mask=valid)
        return carry

    lax.fori_loop(0, NSTEP, step, 0)

    pltpu.sync_copy(acc_c.at[pl.ds(0, NUM_SEG)], shared.at[0, s, 0])
    pltpu.sync_copy(acc_d.at[pl.ds(0, NUM_SEG)], shared.at[1, s, 0])
    pltpu.sync_copy(acc_n.at[pl.ds(0, NUM_SEG)], shared.at[2, s, 0])
    plsc.subcore_barrier()

    sems = (sem_b0, sem_b1)

    def fire(v, b):
        return [
            pltpu.async_copy(
                shared.at[a, v, 0, pl.ds(s * BINS_PER_W, BINS_PER_W)],
                rbuf.at[a, b, 0], sems[b])
            for a in range(3)
        ]

    accs = [[zeros16 for _ in range(BINS_PER_W // 16)] for _ in range(3)]
    pending = fire(0, 0)
    for v in range(NS):
        b = v % 2
        nxt = fire(v + 1, (v + 1) % 2) if v + 1 < NS else None
        for h_ in pending:
            h_.wait()
        for a in range(3):
            for h in range(BINS_PER_W // 16):
                accs[a][h] = accs[a][h] + rbuf[a, b, 0, pl.ds(h * 16, 16)]
        pending = nxt
    for a, res in ((0, res0), (1, res1), (2, res2)):
        for h in range(BINS_PER_W // 16):
            res[pl.ds(h * 16, 16)] = accs[a][h]
    for a, res in ((0, res0), (1, res1), (2, res2)):
        pltpu.sync_copy(
            res,
            out_hbm.at[pl.ds(c * (3 * NUM_SEG) + a * NUM_SEG + s * BINS_PER_W,
                             BINS_PER_W)])


@jax.jit
def _sc_call(lc, ld, ids_pad):
    mesh = plsc.VectorSubcoreMesh(core_axis_name="c", subcore_axis_name="s")
    return pl.kernel(
        _sc_body,
        out_type=jax.ShapeDtypeStruct((NC * 3 * NUM_SEG,), jnp.float32),
        mesh=mesh,
        compiler_params=pltpu.CompilerParams(needs_layout_passes=False),
        scratch_types=[
            pltpu.VMEM((CHUNK,), jnp.float32),
            pltpu.VMEM((CHUNK,), jnp.float32),
            pltpu.VMEM((CHUNK,), jnp.int32),
            pltpu.VMEM((NBIN,), jnp.float32),
            pltpu.VMEM((NBIN,), jnp.float32),
            pltpu.VMEM((NBIN,), jnp.float32),
            pltpu.VMEM((BINS_PER_W,), jnp.float32),
            pltpu.VMEM((BINS_PER_W,), jnp.float32),
            pltpu.VMEM((BINS_PER_W,), jnp.float32),
            pltpu.VMEM_SHARED((3, NS, 1, NUM_SEG), jnp.float32),
            pltpu.VMEM((3, 2, 1, BINS_PER_W), jnp.float32),
            pltpu.SemaphoreType.DMA,
            pltpu.SemaphoreType.DMA,
            pltpu.SemaphoreType.DMA,
        ],
    )(lc, ld, ids_pad)


def _tc_epilogue_body(p_ref, o_ref):
    s0 = p_ref[pl.ds(0, NUM_SEG)] + p_ref[pl.ds(3 * NUM_SEG, NUM_SEG)]
    s1 = p_ref[pl.ds(NUM_SEG, NUM_SEG)] + p_ref[pl.ds(4 * NUM_SEG, NUM_SEG)]
    s2 = p_ref[pl.ds(2 * NUM_SEG, NUM_SEG)] + p_ref[pl.ds(5 * NUM_SEG, NUM_SEG)]
    cnt = jnp.maximum(s2, 1.0)
    o_ref[...] = jnp.stack([s0 / cnt, s1 / cnt])


@jax.jit
def _tc_epilogue(partial):
    return pl.pallas_call(
        _tc_epilogue_body,
        out_shape=jax.ShapeDtypeStruct((2, NUM_SEG), jnp.float32),
    )(partial)


def kernel(t, sigma1, x_pred, x, segment_ids, beta1, one_hot_x, p_0, K):
    ln_s = jnp.log(sigma1[0])
    a = jnp.reshape(-2.0 * ln_s, (1, 1))
    c1 = jnp.reshape(-ln_s, (1, 1))
    c2 = jnp.reshape(K * beta1[0], (1, 1))
    lc, ld = _tc_elem(a, c1, c2, t.T, x_pred.T, x.T, one_hot_x.T, p_0.T)
    partial = _sc_call(lc, ld, segment_ids.astype(jnp.int32))
    return _tc_epilogue(partial)
